# baseline (speedup 1.0000x reference)
import jax
import jax.numpy as jnp
from jax.experimental import pallas as pl

_MAX_REL_POS = 128
_DIM = 64


def _bcast_body(tbl_ref, out_ref):
    i = pl.program_id(0)
    row = tbl_ref[pl.ds(i, 1), :]
    out_ref[...] = jnp.broadcast_to(row[:, None, :], out_ref.shape)


def kernel(time, table):
    _, seq_len = time.shape
    rows = 2 * _MAX_REL_POS + 1
    cols = seq_len * _DIM // 128
    te = jnp.tile(table, (1, 2))
    out = pl.pallas_call(
        _bcast_body,
        grid=(rows,),
        in_specs=[pl.BlockSpec((rows, 128), lambda i: (0, 0))],
        out_specs=pl.BlockSpec((1, cols, 128), lambda i: (i, 0, 0)),
        out_shape=jax.ShapeDtypeStruct((rows, cols, 128), jnp.float32),
    )(te)
    return out.reshape(rows, seq_len, _DIM)


# speedup vs baseline: 1.1954x; 1.1954x over previous
import jax
import jax.numpy as jnp
from jax.experimental import pallas as pl

_MAX_REL_POS = 128
_DIM = 64


_R = 8


def _bcast_body(tbl_ref, out_ref):
    i = pl.program_id(0)
    rows = tbl_ref[pl.ds(i * _R, _R), :]
    out_ref[...] = jnp.broadcast_to(rows[:, None, :], out_ref.shape)


def kernel(time, table):
    _, seq_len = time.shape
    rows = 2 * _MAX_REL_POS + 1
    cols = seq_len * _DIM // 128
    te = jnp.tile(table, (1, 2))
    pad = (-rows) % _R
    te = jnp.pad(te, ((0, pad), (0, 0)))
    grid = (rows + pad) // _R
    out = pl.pallas_call(
        _bcast_body,
        grid=(grid,),
        in_specs=[pl.BlockSpec((rows + pad, 128), lambda i: (0, 0))],
        out_specs=pl.BlockSpec((_R, cols, 128), lambda i: (i, 0, 0)),
        out_shape=jax.ShapeDtypeStruct((rows, cols, 128), jnp.float32),
    )(te)
    return out.reshape(rows, seq_len, _DIM)


# speedup vs baseline: 1.4273x; 1.1940x over previous
import functools

import jax
import jax.numpy as jnp
from jax import lax
from jax.experimental import pallas as pl
from jax.experimental.pallas import tpu as pltpu
from jax.experimental.pallas import tpu_sc as plsc

_MAX_REL_POS = 128
_DIM = 64
_ROWS = 2 * _MAX_REL_POS + 1
_LANES = 16
_REP = 256
_UNROLL = 4


def _build_replicas(buf, rvs, n_copies):
    def body(c, carry):
        base = c * _UNROLL
        for u in range(_UNROLL):
            for k in range(len(rvs)):
                buf[base + u, pl.ds(k * _LANES, _LANES)] = rvs[k]
        return carry

    lax.fori_loop(0, n_copies // _UNROLL, body, 0, unroll=False)


def _sc_body(seq_len, n_workers, rows_per_worker, tbl_hbm, out_hbm,
             tblv, buf0, buf1, sem0, sem1):
    wid = lax.axis_index("s") * 2 + lax.axis_index("c")
    bufs = (buf0, buf1)
    sems = (sem0, sem1)
    n_chunks = seq_len // _REP

    pltpu.sync_copy(tbl_hbm, tblv)

    inflight = [0, 0]
    handles = [[], []]

    for k in range(rows_per_worker):
        i = wid * rows_per_worker + k
        p = k % 2
        for h in handles[p]:
            h.wait()
        handles[p] = []
        rvs = [tblv[i, pl.ds(kk * _LANES, _LANES)] for kk in range(_DIM // _LANES)]
        _build_replicas(bufs[p], rvs, _REP)
        for c in range(n_chunks):
            handles[p].append(
                pltpu.async_copy(bufs[p], out_hbm.at[i, pl.ds(c * _REP, _REP)],
                                 sems[p]))

    first_left = n_workers * rows_per_worker
    n_left = _ROWS - first_left
    seq_per_worker = seq_len // n_workers
    for m in range(n_left):
        i = first_left + m
        p = (rows_per_worker + m) % 2
        for h in handles[p]:
            h.wait()
        handles[p] = []
        rvs = [tblv[i, pl.ds(kk * _LANES, _LANES)] for kk in range(_DIM // _LANES)]
        _build_replicas(bufs[p], rvs, seq_per_worker)
        handles[p].append(
            pltpu.async_copy(bufs[p].at[pl.ds(0, seq_per_worker)],
                             out_hbm.at[i, pl.ds(wid * seq_per_worker,
                                                 seq_per_worker)],
                             sems[p]))

    for p in range(2):
        for h in handles[p]:
            h.wait()


def kernel(time, table):
    _, seq_len = time.shape
    n_workers = 32
    rows_per_worker = _ROWS // n_workers
    assert seq_len % _REP == 0 and seq_len % n_workers == 0
    assert _REP % _UNROLL == 0 and (seq_len // n_workers) % _UNROLL == 0

    mesh = plsc.VectorSubcoreMesh(core_axis_name="c", subcore_axis_name="s")
    body = functools.partial(_sc_body, seq_len, n_workers, rows_per_worker)
    f = pl.kernel(
        body,
        out_type=jax.ShapeDtypeStruct((_ROWS, seq_len, _DIM), jnp.float32),
        mesh=mesh,
        scratch_types=[
            pltpu.VMEM((_ROWS, _DIM), jnp.float32),
            pltpu.VMEM((_REP, _DIM), jnp.float32),
            pltpu.VMEM((_REP, _DIM), jnp.float32),
            pltpu.SemaphoreType.DMA,
            pltpu.SemaphoreType.DMA,
        ],
    )
    return f(table)


# speedup vs baseline: 1.4285x; 1.0008x over previous
import functools

import jax
import jax.numpy as jnp
from jax import lax
from jax.experimental import pallas as pl
from jax.experimental.pallas import tpu as pltpu
from jax.experimental.pallas import tpu_sc as plsc

_MAX_REL_POS = 128
_DIM = 64
_ROWS = 2 * _MAX_REL_POS + 1
_LANES = 16
_REP = 256
_UNROLL = 4


def _build_replicas(buf, rvs, n_copies):
    def body(c, carry):
        base = c * _UNROLL
        for u in range(_UNROLL):
            for k in range(len(rvs)):
                buf[base + u, pl.ds(k * _LANES, _LANES)] = rvs[k]
        return carry

    lax.fori_loop(0, n_copies // _UNROLL, body, 0, unroll=False)


def _sc_body(seq_len, n_workers, rows_per_worker, tbl_hbm, out_hbm,
             tblv, buf0, buf1, sem0, sem1):
    wid = lax.axis_index("s") * 2 + lax.axis_index("c")
    bufs = (buf0, buf1)
    sems = (sem0, sem1)
    n_chunks = seq_len // _REP

    pltpu.sync_copy(tbl_hbm, tblv)

    inflight = [0, 0]
    handles = [[], []]

    for k in range(rows_per_worker):
        i = wid * rows_per_worker + k
        p = k % 2
        for h in handles[p]:
            h.wait()
        handles[p] = []
        rvs = [tblv[i, pl.ds(kk * _LANES, _LANES)] for kk in range(_DIM // _LANES)]
        _build_replicas(bufs[p], rvs, _REP)
        for c in range(n_chunks):
            handles[p].append(
                pltpu.async_copy(bufs[p], out_hbm.at[i, pl.ds(c * _REP, _REP)],
                                 sems[p]))

    first_left = n_workers * rows_per_worker
    n_left = _ROWS - first_left
    seq_per_worker = seq_len // n_workers
    for m in range(n_left):
        i = first_left + m
        p = (rows_per_worker + m) % 2
        for h in handles[p]:
            h.wait()
        handles[p] = []
        rvs = [tblv[i, pl.ds(kk * _LANES, _LANES)] for kk in range(_DIM // _LANES)]
        _build_replicas(bufs[p], rvs, seq_per_worker)
        handles[p].append(
            pltpu.async_copy(bufs[p].at[pl.ds(0, seq_per_worker)],
                             out_hbm.at[i, pl.ds(wid * seq_per_worker,
                                                 seq_per_worker)],
                             sems[p]))

    for p in range(2):
        for h in handles[p]:
            h.wait()


def kernel(time, table):
    _, seq_len = time.shape
    n_workers = 32
    rows_per_worker = _ROWS // n_workers
    assert seq_len % _REP == 0 and seq_len % n_workers == 0
    assert _REP % _UNROLL == 0 and (seq_len // n_workers) % _UNROLL == 0

    mesh = plsc.VectorSubcoreMesh(core_axis_name="c", subcore_axis_name="s")
    body = functools.partial(_sc_body, seq_len, n_workers, rows_per_worker)
    f = pl.kernel(
        body,
        out_type=jax.ShapeDtypeStruct((_ROWS, seq_len, _DIM), jnp.float32),
        mesh=mesh,
        scratch_types=[
            pltpu.VMEM((_ROWS, _DIM), jnp.float32),
            pltpu.VMEM((_REP, _DIM), jnp.float32),
            pltpu.VMEM((_REP, _DIM), jnp.float32),
            pltpu.SemaphoreType.DMA,
            pltpu.SemaphoreType.DMA,
        ],
        compiler_params=pltpu.CompilerParams(use_tc_tiling_on_sc=True),
    )
    return f(table)


# speedup vs baseline: 5.9123x; 4.1390x over previous
import functools

import jax
import jax.numpy as jnp
from jax import lax
from jax.experimental import pallas as pl
from jax.experimental.pallas import tpu as pltpu
from jax.experimental.pallas import tpu_sc as plsc

_MAX_REL_POS = 128
_DIM = 64
_ROWS = 2 * _MAX_REL_POS + 1
_LANES = 16
_JB = 512


def _build_block(buf, tblv, i):
    def dbody(d, carry):
        v = plsc.load_gather(tblv, [jnp.full((_LANES,), i, jnp.int32),
                                    jnp.full((_LANES,), d, jnp.int32)])
        for c in range(_JB // _LANES):
            buf[d, pl.ds(c * _LANES, _LANES)] = v
        return carry

    lax.fori_loop(0, _DIM, dbody, 0)


def _sc_body(seq_len, n_workers, rows_per_worker, tbl_hbm, out_hbm,
             tblv, buf0, buf1, sem0, sem1):
    wid = lax.axis_index("s") * 2 + lax.axis_index("c")
    bufs = (buf0, buf1)
    sems = (sem0, sem1)
    n_chunks = seq_len // _JB

    pltpu.sync_copy(tbl_hbm, tblv)

    handles = [[], []]

    for k in range(rows_per_worker):
        i = wid * rows_per_worker + k
        p = k % 2
        for h in handles[p]:
            h.wait()
        handles[p] = []
        _build_block(bufs[p], tblv, i)
        for c in range(n_chunks):
            handles[p].append(
                pltpu.async_copy(bufs[p],
                                 out_hbm.at[i, :, pl.ds(c * _JB, _JB)],
                                 sems[p]))

    first_left = n_workers * rows_per_worker
    n_left = _ROWS - first_left
    for p in range(2):
        for h in handles[p]:
            h.wait()
        handles[p] = []
    for m in range(n_left):
        i = first_left + m
        jb = _JB // 2
        n_sl = seq_len // jb

        @pl.when(wid < n_sl)
        def _():
            _build_block(bufs[0], tblv, i)
            pltpu.async_copy(bufs[0].at[:, pl.ds(0, jb)],
                             out_hbm.at[i, :, pl.ds(wid * jb, jb)],
                             sems[0]).wait()


def kernel(time, table):
    _, seq_len = time.shape
    n_workers = 32
    rows_per_worker = _ROWS // n_workers
    assert seq_len % _JB == 0

    mesh = plsc.VectorSubcoreMesh(core_axis_name="c", subcore_axis_name="s")
    body = functools.partial(_sc_body, seq_len, n_workers, rows_per_worker)
    f = pl.kernel(
        body,
        out_type=jax.ShapeDtypeStruct((_ROWS, _DIM, seq_len), jnp.float32),
        mesh=mesh,
        scratch_types=[
            pltpu.VMEM((_ROWS, _DIM), jnp.float32),
            pltpu.VMEM((_DIM, _JB), jnp.float32),
            pltpu.VMEM((_DIM, _JB), jnp.float32),
            pltpu.SemaphoreType.DMA,
            pltpu.SemaphoreType.DMA,
        ],
        compiler_params=pltpu.CompilerParams(use_tc_tiling_on_sc=True,
                                             needs_layout_passes=False),
    )
    out = f(table)
    return jnp.swapaxes(out, 1, 2)


# speedup vs baseline: 6.2457x; 1.0564x over previous
import functools

import jax
import jax.numpy as jnp
from jax import lax
from jax.experimental import pallas as pl
from jax.experimental.pallas import tpu as pltpu
from jax.experimental.pallas import tpu_sc as plsc

_MAX_REL_POS = 128
_DIM = 64
_ROWS = 2 * _MAX_REL_POS + 1
_LANES = 16
_JB = 512
_NBUF = 3


def _build_block(buf, tblw, k):
    def dbody(d, carry):
        v = plsc.load_gather(tblw, [jnp.full((_LANES,), k, jnp.int32),
                                    jnp.full((_LANES,), d, jnp.int32)])
        for c in range(_JB // _LANES):
            buf[d, pl.ds(c * _LANES, _LANES)] = v
        return carry

    lax.fori_loop(0, _DIM, dbody, 0)


def _sc_body(seq_len, n_workers, rows_per_worker, tbl_hbm, out_hbm,
             tblw, buf0, buf1, buf2, sem0, sem1, sem2):
    wid = lax.axis_index("s") * 2 + lax.axis_index("c")
    bufs = (buf0, buf1, buf2)
    sems = (sem0, sem1, sem2)
    n_chunks = seq_len // _JB
    base = wid * rows_per_worker

    pltpu.sync_copy(tbl_hbm.at[pl.ds(base, rows_per_worker)], tblw)

    handles = [[], [], []]

    for k in range(rows_per_worker):
        p = k % _NBUF
        for h in handles[p]:
            h.wait()
        handles[p] = []
        _build_block(bufs[p], tblw, k)
        for c in range(n_chunks):
            handles[p].append(
                pltpu.async_copy(bufs[p],
                                 out_hbm.at[base + k, :,
                                            pl.ds(c * _JB, _JB)],
                                 sems[p]))

    first_left = n_workers * rows_per_worker
    n_left = _ROWS - first_left
    jb = 128
    n_sl = seq_len // jb
    for m in range(n_left):
        p = (rows_per_worker + m) % _NBUF
        for h in handles[p]:
            h.wait()
        handles[p] = []

        @pl.when(wid < n_sl)
        def _():
            pltpu.sync_copy(tbl_hbm.at[pl.ds(first_left + m, 8)],
                            tblw)
            _build_block(bufs[p], tblw, 0)
            pltpu.async_copy(bufs[p].at[:, pl.ds(0, jb)],
                             out_hbm.at[first_left + m, :,
                                        pl.ds(wid * jb, jb)],
                             sems[p]).wait()

    for p in range(_NBUF):
        for h in handles[p]:
            h.wait()


def kernel(time, table):
    _, seq_len = time.shape
    n_workers = 32
    rows_per_worker = _ROWS // n_workers
    assert seq_len % _JB == 0 and seq_len % 128 == 0

    te = jnp.pad(table, ((0, (-_ROWS) % 8), (0, 0)))

    mesh = plsc.VectorSubcoreMesh(core_axis_name="c", subcore_axis_name="s")
    body = functools.partial(_sc_body, seq_len, n_workers, rows_per_worker)
    f = pl.kernel(
        body,
        out_type=jax.ShapeDtypeStruct((_ROWS, _DIM, seq_len), jnp.float32),
        mesh=mesh,
        scratch_types=[
            pltpu.VMEM((rows_per_worker, _DIM), jnp.float32),
            pltpu.VMEM((_DIM, _JB), jnp.float32),
            pltpu.VMEM((_DIM, _JB), jnp.float32),
            pltpu.VMEM((_DIM, _JB), jnp.float32),
            pltpu.SemaphoreType.DMA,
            pltpu.SemaphoreType.DMA,
            pltpu.SemaphoreType.DMA,
        ],
        compiler_params=pltpu.CompilerParams(use_tc_tiling_on_sc=True,
                                             needs_layout_passes=False),
    )
    out = f(te)
    return jnp.swapaxes(out, 1, 2)
